# baseline (speedup 1.0000x reference)
import jax
import jax.numpy as jnp
from jax.experimental import pallas as pl
from jax.experimental.pallas import tpu as pltpu

_M = 512
_K = 8193
_N = 2049
_KM = 8192
_NT = 416
_KC = 4
_KW = _KM // _KC


def _mm_body(*refs):
    wt_refs = refs[:_KC]
    xt_refs = refs[_KC:2 * _KC]
    wl_ref, xl_ref, o_ref, xs_ref = refs[2 * _KC:]

    @pl.when(pl.program_id(0) == 0)
    def _cast_x():
        for c in range(_KC):
            xs_ref[c * _KW:(c + 1) * _KW, :] = (
                xt_refs[c][...].astype(jnp.bfloat16))

    acc = wl_ref[...] * xl_ref[...]
    for c in range(_KC):
        wb = wt_refs[c][...].astype(jnp.bfloat16)
        acc += jax.lax.dot_general(
            wb, xs_ref[c * _KW:(c + 1) * _KW, :], (((1,), (0,)), ((), ())),
            preferred_element_type=jnp.float32)
    o_ref[...] = acc


def kernel(x_affine, W):
    xt = x_affine.T
    wt = W.T
    wl = jax.lax.slice(wt, (0, _KM), (_N, _K))
    xl = jax.lax.slice(xt, (_KM, 0), (_K, _M))
    wt_specs = [
        pl.BlockSpec((_NT, _KW), lambda j, c=c: (j, c)) for c in range(_KC)
    ]
    xt_specs = [
        pl.BlockSpec((_KW, _M), lambda j, c=c: (c, 0)) for c in range(_KC)
    ]
    yt = pl.pallas_call(
        _mm_body,
        grid=(pl.cdiv(_N, _NT),),
        in_specs=wt_specs + xt_specs + [
            pl.BlockSpec((_NT, 1), lambda j: (j, 0)),
            pl.BlockSpec((1, _M), lambda j: (0, 0)),
        ],
        out_specs=pl.BlockSpec((_NT, _M), lambda j: (j, 0)),
        out_shape=jax.ShapeDtypeStruct((_N, _M), jnp.float32),
        scratch_shapes=[pltpu.VMEM((_KM, _M), jnp.bfloat16)],
    )(*([wt] * _KC + [xt] * _KC + [wl, xl]))
    return yt.T


# speedup vs baseline: 1.0696x; 1.0696x over previous
import jax
import jax.numpy as jnp
from jax.experimental import pallas as pl
from jax.experimental.pallas import tpu as pltpu

_M = 512
_K = 8193
_N = 2049
_KM = 8192
_NT = 416
_KC = 4
_KW = _KM // _KC


def _mm_body(wt_ref, *refs):
    xt_refs = refs[:_KC]
    xl_ref, o_ref, xs_ref = refs[_KC:]

    @pl.when(pl.program_id(0) == 0)
    def _cast_x():
        for c in range(_KC):
            xs_ref[c * _KW:(c + 1) * _KW, :] = (
                xt_refs[c][...].astype(jnp.bfloat16))

    acc = wt_ref[:, _KM:] * xl_ref[...]
    for c in range(_KC):
        wb = wt_ref[:, c * _KW:(c + 1) * _KW].astype(jnp.bfloat16)
        acc += jax.lax.dot_general(
            wb, xs_ref[c * _KW:(c + 1) * _KW, :], (((1,), (0,)), ((), ())),
            preferred_element_type=jnp.float32)
    o_ref[...] = acc


def kernel(x_affine, W):
    xt = x_affine.T
    wt = W.T
    xl = jax.lax.slice(xt, (_KM, 0), (_K, _M))
    xt_specs = [
        pl.BlockSpec((_KW, _M), lambda j, c=c: (c, 0)) for c in range(_KC)
    ]
    yt = pl.pallas_call(
        _mm_body,
        grid=(pl.cdiv(_N, _NT),),
        in_specs=[pl.BlockSpec((_NT, _K), lambda j: (j, 0))] + xt_specs + [
            pl.BlockSpec((1, _M), lambda j: (0, 0)),
        ],
        out_specs=pl.BlockSpec((_NT, _M), lambda j: (j, 0)),
        out_shape=jax.ShapeDtypeStruct((_N, _M), jnp.float32),
        scratch_shapes=[pltpu.VMEM((_KM, _M), jnp.bfloat16)],
    )(*([wt] + [xt] * _KC + [xl]))
    return yt.T


# speedup vs baseline: 1.1074x; 1.0353x over previous
import jax
import jax.numpy as jnp
from jax.experimental import pallas as pl
from jax.experimental.pallas import tpu as pltpu

_M = 512
_K = 8193
_N = 2049
_KM = 8192
_NT = 416
_KC = 8
_KW = _KM // _KC


def _mm_body(wt_ref, xt_ref, o_ref, xs_ref):
    @pl.when(pl.program_id(0) == 0)
    def _cast_x():
        xs_ref[...] = xt_ref[:_KM, :].astype(jnp.bfloat16)

    acc = wt_ref[:, _KM:] * xt_ref[_KM:, :]
    for c in range(_KC):
        wb = wt_ref[:, c * _KW:(c + 1) * _KW].astype(jnp.bfloat16)
        acc += jax.lax.dot_general(
            wb, xs_ref[c * _KW:(c + 1) * _KW, :], (((1,), (0,)), ((), ())),
            preferred_element_type=jnp.float32)
    o_ref[...] = acc


def kernel(x_affine, W):
    xt = x_affine.T
    wt = W.T
    yt = pl.pallas_call(
        _mm_body,
        grid=(pl.cdiv(_N, _NT),),
        in_specs=[
            pl.BlockSpec((_NT, _K), lambda j: (j, 0)),
            pl.BlockSpec((_K, _M), lambda j: (0, 0)),
        ],
        out_specs=pl.BlockSpec((_NT, _M), lambda j: (j, 0)),
        out_shape=jax.ShapeDtypeStruct((_N, _M), jnp.float32),
        scratch_shapes=[pltpu.VMEM((_KM, _M), jnp.bfloat16)],
    )(wt, xt)
    return yt.T


# speedup vs baseline: 1.1259x; 1.0168x over previous
import jax
import jax.numpy as jnp
from jax.experimental import pallas as pl
from jax.experimental.pallas import tpu as pltpu

_M = 512
_K = 8193
_N = 2049
_KM = 8192
_NT = 344
_KC = 8
_KW = _KM // _KC


def _mm_body(wt_ref, xt_ref, o_ref, xs_ref):
    @pl.when(pl.program_id(0) == 0)
    def _cast_x():
        xs_ref[...] = xt_ref[:_KM, :].astype(jnp.bfloat16)

    acc = wt_ref[:, _KM:] * xt_ref[_KM:, :]
    for c in range(_KC):
        wb = wt_ref[:, c * _KW:(c + 1) * _KW].astype(jnp.bfloat16)
        acc += jax.lax.dot_general(
            wb, xs_ref[c * _KW:(c + 1) * _KW, :], (((1,), (0,)), ((), ())),
            preferred_element_type=jnp.float32)
    o_ref[...] = acc


def kernel(x_affine, W):
    xt = x_affine.T
    wt = W.T
    yt = pl.pallas_call(
        _mm_body,
        grid=(pl.cdiv(_N, _NT),),
        in_specs=[
            pl.BlockSpec((_NT, _K), lambda j: (j, 0)),
            pl.BlockSpec((_K, _M), lambda j: (0, 0)),
        ],
        out_specs=pl.BlockSpec((_NT, _M), lambda j: (j, 0)),
        out_shape=jax.ShapeDtypeStruct((_N, _M), jnp.float32),
        scratch_shapes=[pltpu.VMEM((_KM, _M), jnp.bfloat16)],
    )(wt, xt)
    return yt.T
